# Initial kernel scaffold; baseline (speedup 1.0000x reference)
import jax
import jax.numpy as jnp
from jax.experimental import pallas as pl


def kernel(output):
    raise NotImplementedError("write your pallas kernel here")


# speedup vs baseline: 9.6890x; 9.6890x over previous
import numpy as np
import jax
import jax.numpy as jnp
from jax.experimental import pallas as pl

_ANCHORS = np.array([1.3221, 1.73145, 3.19275, 4.00944, 5.05587, 8.09892,
                     9.47112, 4.84053, 11.2364, 10.0071], dtype=np.float32)
_A = 5
_CLS = 20
_G = 19
_N = _A * _G * _G
_NP = 1920
_B = 8
_CONF_TH = 0.005
_NMS_TH = 0.45


def _make_consts():
    i = np.arange(_NP)
    a = np.minimum(i // (_G * _G), _A - 1)
    p = i % (_G * _G)
    gx = (p % _G).astype(np.float32)
    gy = (p // _G).astype(np.float32)
    aw = _ANCHORS[2 * a]
    ah = _ANCHORS[2 * a + 1]
    return np.stack([gx, gy, aw, ah]).reshape(4, 1, _NP).astype(np.float32)


_CONSTS = _make_consts()


def _nms_body(o_ref, c_ref, cx_ref, cy_ref, w_ref, h_ref, p_ref):
    gx = c_ref[0]
    gy = c_ref[1]
    aw = c_ref[2]
    ah = c_ref[3]

    cx = (jax.nn.sigmoid(o_ref[0]) + gx) / np.float32(_G)
    cy = (jax.nn.sigmoid(o_ref[1]) + gy) / np.float32(_G)
    wv = jnp.exp(o_ref[2]) * aw / np.float32(_G)
    hv = jnp.exp(o_ref[3]) * ah / np.float32(_G)
    conf = jax.nn.sigmoid(o_ref[4])

    maxl = o_ref[5]
    for c in range(6, 5 + _CLS):
        maxl = jnp.maximum(maxl, o_ref[c])
    ssum = jnp.exp(o_ref[5] - maxl)
    for c in range(6, 5 + _CLS):
        ssum = ssum + jnp.exp(o_ref[c] - maxl)
    clsc = 1.0 / ssum

    half_w = wv / 2.0
    half_h = hv / 2.0
    x1 = cx - half_w
    x2 = cx + half_w
    y1 = cy - half_h
    y2 = cy + half_h
    area = (x2 - x1) * (y2 - y1)

    lane = jax.lax.broadcasted_iota(jnp.int32, (_B, _NP), 1)
    valid = lane < _N
    alive0 = ((conf > _CONF_TH) & valid).astype(jnp.float32)
    kept0 = jnp.zeros((_B, _NP), jnp.float32)
    neg = np.float32(-3.4e38)

    def cond(st):
        a, _ = st
        return jnp.max(a) > 0.0

    def body(st):
        af, kf = st
        a = af > 0.0
        masked = jnp.where(a, conf, -1.0)
        m = jnp.max(masked, axis=1, keepdims=True)
        ismax = (masked == m) & a
        li = jnp.min(jnp.where(ismax, lane, np.int32(2 ** 30)),
                     axis=1, keepdims=True)
        leader = lane == li

        def ext(v):
            return jnp.max(jnp.where(leader, v, neg), axis=1, keepdims=True)

        lx1 = ext(x1)
        lx2 = ext(x2)
        ly1 = ext(y1)
        ly2 = ext(y2)
        la = ext(area)
        iw = jnp.maximum(jnp.minimum(x2, lx2) - jnp.maximum(x1, lx1), 0.0)
        ih = jnp.maximum(jnp.minimum(y2, ly2) - jnp.maximum(y1, ly1), 0.0)
        inter = iw * ih
        iou = inter / jnp.maximum(area + la - inter, np.float32(1e-12))
        sup = a & (iou > _NMS_TH)
        new_a = a & jnp.logical_not(sup) & jnp.logical_not(leader)
        return new_a.astype(jnp.float32), jnp.maximum(kf, leader.astype(jnp.float32))

    _, kept = jax.lax.while_loop(cond, body, (alive0, kept0))

    prob = conf * clsc * kept
    cx_ref[:, :] = cx
    cy_ref[:, :] = cy
    w_ref[:, :] = wv
    h_ref[:, :] = hv
    p_ref[:, :] = prob


def kernel(output):
    o = jnp.transpose(output.reshape(_B, _A, 5 + _CLS, _G * _G),
                      (2, 0, 1, 3)).reshape(5 + _CLS, _B, _N)
    o = jnp.pad(o, ((0, 0), (0, 0), (0, _NP - _N)))
    outs = pl.pallas_call(
        _nms_body,
        out_shape=[jax.ShapeDtypeStruct((_B, _NP), jnp.float32)] * 5,
    )(o, jnp.asarray(_CONSTS))
    return jnp.stack(outs, axis=-1)[:, :_N, :]


# speedup vs baseline: 11.0335x; 1.1388x over previous
import numpy as np
import jax
import jax.numpy as jnp
from jax.experimental import pallas as pl

_ANCHORS = np.array([1.3221, 1.73145, 3.19275, 4.00944, 5.05587, 8.09892,
                     9.47112, 4.84053, 11.2364, 10.0071], dtype=np.float32)
_A = 5
_CLS = 20
_G = 19
_N = _A * _G * _G
_NP = 1920
_B = 8
_CONF_TH = 0.005
_NMS_TH = 0.45


def _make_consts():
    i = np.arange(_NP)
    a = np.minimum(i // (_G * _G), _A - 1)
    p = i % (_G * _G)
    gx = (p % _G).astype(np.float32)
    gy = (p // _G).astype(np.float32)
    aw = _ANCHORS[2 * a]
    ah = _ANCHORS[2 * a + 1]
    return np.stack([gx, gy, aw, ah]).reshape(4, 1, _NP).astype(np.float32)


_CONSTS = _make_consts()


def _nms_body(o_ref, c_ref, cx_ref, cy_ref, w_ref, h_ref, p_ref):
    gx = c_ref[0]
    gy = c_ref[1]
    aw = c_ref[2]
    ah = c_ref[3]

    cx = (jax.nn.sigmoid(o_ref[0]) + gx) / np.float32(_G)
    cy = (jax.nn.sigmoid(o_ref[1]) + gy) / np.float32(_G)
    wv = jnp.exp(o_ref[2]) * aw / np.float32(_G)
    hv = jnp.exp(o_ref[3]) * ah / np.float32(_G)
    conf = jax.nn.sigmoid(o_ref[4])

    maxl = o_ref[5]
    for c in range(6, 5 + _CLS):
        maxl = jnp.maximum(maxl, o_ref[c])
    ssum = jnp.exp(o_ref[5] - maxl)
    for c in range(6, 5 + _CLS):
        ssum = ssum + jnp.exp(o_ref[c] - maxl)
    clsc = 1.0 / ssum

    half_w = wv / 2.0
    half_h = hv / 2.0
    x1 = cx - half_w
    x2 = cx + half_w
    y1 = cy - half_h
    y2 = cy + half_h
    sa = (x2 - x1) * (y2 - y1) * np.float32(_NMS_TH / (1.0 + _NMS_TH))

    lane = jax.lax.broadcasted_iota(jnp.int32, (_B, _NP), 1)
    valid = lane < _N
    alive0 = ((conf > _CONF_TH) & valid).astype(jnp.float32)
    kept0 = jnp.zeros((_B, _NP), jnp.float32)
    neg = np.float32(-3.4e38)

    def cond(st):
        a, _ = st
        return jnp.max(a) > 0.0

    def step(af, kf):
        a = af > 0.0
        masked = jnp.where(a, conf, -1.0)
        m = jnp.max(masked, axis=1, keepdims=True)
        ismax = (masked == m) & a
        li = jnp.min(jnp.where(ismax, lane, np.int32(2 ** 30)),
                     axis=1, keepdims=True)
        leader = lane == li

        def ext(v):
            return jnp.max(jnp.where(leader, v, neg), axis=1, keepdims=True)

        lcx = ext(cx)
        lcy = ext(cy)
        lw = ext(wv)
        lh = ext(hv)
        lx1 = lcx - lw / 2.0
        lx2 = lcx + lw / 2.0
        ly1 = lcy - lh / 2.0
        ly2 = lcy + lh / 2.0
        lsa = (lx2 - lx1) * (ly2 - ly1) * np.float32(_NMS_TH / (1.0 + _NMS_TH))
        iw = jnp.maximum(jnp.minimum(x2, lx2) - jnp.maximum(x1, lx1), 0.0)
        ih = jnp.maximum(jnp.minimum(y2, ly2) - jnp.maximum(y1, ly1), 0.0)
        sup = a & (iw * ih > sa + lsa)
        new_a = a & jnp.logical_not(sup) & jnp.logical_not(leader)
        return new_a.astype(jnp.float32), jnp.maximum(kf, leader.astype(jnp.float32))

    def body(st):
        af, kf = st
        af, kf = step(af, kf)
        af, kf = step(af, kf)
        return af, kf

    _, kept = jax.lax.while_loop(cond, body, (alive0, kept0))

    prob = conf * clsc * kept
    cx_ref[:, :] = cx
    cy_ref[:, :] = cy
    w_ref[:, :] = wv
    h_ref[:, :] = hv
    p_ref[:, :] = prob


def kernel(output):
    o = jnp.transpose(output.reshape(_B, _A, 5 + _CLS, _G * _G),
                      (2, 0, 1, 3)).reshape(5 + _CLS, _B, _N)
    o = jnp.pad(o, ((0, 0), (0, 0), (0, _NP - _N)))
    outs = pl.pallas_call(
        _nms_body,
        out_shape=[jax.ShapeDtypeStruct((_B, _NP), jnp.float32)] * 5,
    )(o, jnp.asarray(_CONSTS))
    return jnp.stack(outs, axis=-1)[:, :_N, :]


# speedup vs baseline: 12.1121x; 1.0978x over previous
import numpy as np
import jax
import jax.numpy as jnp
from jax.experimental import pallas as pl

_ANCHORS = np.array([1.3221, 1.73145, 3.19275, 4.00944, 5.05587, 8.09892,
                     9.47112, 4.84053, 11.2364, 10.0071], dtype=np.float32)
_A = 5
_CLS = 20
_G = 19
_N = _A * _G * _G
_NP = 1920
_B = 8
_CONF_TH = 0.005
_NMS_TH = 0.45


def _make_consts():
    i = np.arange(_NP)
    a = np.minimum(i // (_G * _G), _A - 1)
    p = i % (_G * _G)
    gx = (p % _G).astype(np.float32)
    gy = (p // _G).astype(np.float32)
    aw = _ANCHORS[2 * a]
    ah = _ANCHORS[2 * a + 1]
    return np.stack([gx, gy, aw, ah]).reshape(4, 1, _NP).astype(np.float32)


_CONSTS = _make_consts()


def _nms_body(o_ref, c_ref, cx_ref, cy_ref, w_ref, h_ref, p_ref):
    gx = c_ref[0]
    gy = c_ref[1]
    aw = c_ref[2]
    ah = c_ref[3]

    cx = (jax.nn.sigmoid(o_ref[0]) + gx) / np.float32(_G)
    cy = (jax.nn.sigmoid(o_ref[1]) + gy) / np.float32(_G)
    wv = jnp.exp(o_ref[2]) * aw / np.float32(_G)
    hv = jnp.exp(o_ref[3]) * ah / np.float32(_G)
    conf = jax.nn.sigmoid(o_ref[4])

    maxl = o_ref[5]
    for c in range(6, 5 + _CLS):
        maxl = jnp.maximum(maxl, o_ref[c])
    ssum = jnp.exp(o_ref[5] - maxl)
    for c in range(6, 5 + _CLS):
        ssum = ssum + jnp.exp(o_ref[c] - maxl)
    clsc = 1.0 / ssum

    half_w = wv / 2.0
    half_h = hv / 2.0
    x1 = cx - half_w
    x2 = cx + half_w
    y1 = cy - half_h
    y2 = cy + half_h
    sa = (x2 - x1) * (y2 - y1) * np.float32(_NMS_TH / (1.0 + _NMS_TH))

    lane = jax.lax.broadcasted_iota(jnp.int32, (_B, _NP), 1)
    valid = lane < _N
    alive0 = ((conf > _CONF_TH) & valid).astype(jnp.float32)
    kept0 = jnp.zeros((_B, _NP), jnp.float32)
    neg = np.float32(-3.4e38)

    def cond(st):
        a, _ = st
        return jnp.max(a) > 0.0

    def step(af, kf):
        a = af > 0.0
        masked = jnp.where(a, conf, -1.0)
        m = jnp.max(masked, axis=1, keepdims=True)
        ismax = (masked == m) & a
        li = jnp.min(jnp.where(ismax, lane, np.int32(2 ** 30)),
                     axis=1, keepdims=True)
        leader = lane == li

        def ext(v):
            return jnp.max(jnp.where(leader, v, neg), axis=1, keepdims=True)

        lcx = ext(cx)
        lcy = ext(cy)
        lw = ext(wv)
        lh = ext(hv)
        lx1 = lcx - lw / 2.0
        lx2 = lcx + lw / 2.0
        ly1 = lcy - lh / 2.0
        ly2 = lcy + lh / 2.0
        lsa = (lx2 - lx1) * (ly2 - ly1) * np.float32(_NMS_TH / (1.0 + _NMS_TH))
        iw = jnp.maximum(jnp.minimum(x2, lx2) - jnp.maximum(x1, lx1), 0.0)
        ih = jnp.maximum(jnp.minimum(y2, ly2) - jnp.maximum(y1, ly1), 0.0)
        sup = a & (iw * ih > sa + lsa)
        new_a = a & jnp.logical_not(sup) & jnp.logical_not(leader)
        return new_a.astype(jnp.float32), jnp.maximum(kf, leader.astype(jnp.float32))

    def body(st):
        af, kf = st
        for _ in range(8):
            af, kf = step(af, kf)
        return af, kf

    _, kept = jax.lax.while_loop(cond, body, (alive0, kept0))

    prob = conf * clsc * kept
    cx_ref[:, :] = cx
    cy_ref[:, :] = cy
    w_ref[:, :] = wv
    h_ref[:, :] = hv
    p_ref[:, :] = prob


def kernel(output):
    o = jnp.transpose(output.reshape(_B, _A, 5 + _CLS, _G * _G),
                      (2, 0, 1, 3)).reshape(5 + _CLS, _B, _N)
    o = jnp.pad(o, ((0, 0), (0, 0), (0, _NP - _N)))
    outs = pl.pallas_call(
        _nms_body,
        out_shape=[jax.ShapeDtypeStruct((_B, _NP), jnp.float32)] * 5,
    )(o, jnp.asarray(_CONSTS))
    return jnp.stack(outs, axis=-1)[:, :_N, :]
